# baseline (speedup 1.0000x reference)
import jax
import jax.numpy as jnp
from jax import lax
from jax.experimental import pallas as pl
from jax.experimental.pallas import tpu as pltpu
from jax.experimental.pallas import tpu_sc as plsc

N = 100000
NP = 100096
E = 1600000
NC, NS = 2, 16
NW = NC * NS
EPW = E // NW
RING = 3
CB = 2560
_CHUNKS = []
_off = 0
while _off < EPW:
    _sz = min(CB, EPW - _off)
    _CHUNKS.append((_off, _sz))
    _off += _sz
NCHUNK = len(_CHUNKS)

PCOEF = 917.0 * 9.81
BCOEF = 1000.0 * 9.81


def _unroll(trips):
    for u in (8, 5, 4, 3, 2):
        if trips % u == 0:
            return u
    return 1


def _phi_body(h_ref, b_ref, o_ref):
    o_ref[...] = PCOEF * h_ref[...] + BCOEF * b_ref[...]


def _sc_body(phi_hbm, head_hbm, tail_hbm, len_hbm, out_hbm, phi_v, *rest):
    c = lax.axis_index("c")
    s = lax.axis_index("s")
    wid = s * NC + c
    base = wid * EPW
    bufs = tuple(rest[4 * r:4 * r + 4] + (rest[4 * RING + 2 * r], rest[4 * RING + 2 * r + 1])
                 for r in range(RING))

    def fire_in(k):
        h, t, l, _, sin, _ = bufs[k % RING]
        off, sz = _CHUNKS[k]
        st = base + off
        return (
            pltpu.async_copy(head_hbm.at[pl.ds(st, sz)], h.at[pl.ds(0, sz)], sin),
            pltpu.async_copy(tail_hbm.at[pl.ds(st, sz)], t.at[pl.ds(0, sz)], sin),
            pltpu.async_copy(len_hbm.at[pl.ds(st, sz)], l.at[pl.ds(0, sz)], sin),
        )

    in_flight = {k: fire_in(k) for k in range(min(RING, NCHUNK))}
    pltpu.sync_copy(phi_hbm, phi_v)
    out_flight = {}
    for k in range(NCHUNK):
        h, t, l, o, _, sout = bufs[k % RING]
        off, sz = _CHUNKS[k]
        for cdesc in in_flight.pop(k):
            cdesc.wait()
        if k - RING in out_flight:
            out_flight.pop(k - RING).wait()

        @plsc.parallel_loop(0, sz, step=16, unroll=_unroll(sz // 16))
        def _gather(i):
            sl = pl.ds(i, 16)
            ph = plsc.load_gather(phi_v, [h[sl]])
            pt = plsc.load_gather(phi_v, [t[sl]])
            o[sl] = (pt - ph) / l[sl]

        st = base + off
        out_flight[k] = pltpu.async_copy(o.at[pl.ds(0, sz)], out_hbm.at[pl.ds(st, sz)], sout)
        if k + RING < NCHUNK:
            in_flight[k + RING] = fire_in(k + RING)
    for cdesc in out_flight.values():
        cdesc.wait()


_sc_call = pl.kernel(
    _sc_body,
    out_type=jax.ShapeDtypeStruct((E,), jnp.float32),
    mesh=plsc.VectorSubcoreMesh(core_axis_name="c", subcore_axis_name="s"),
    compiler_params=pltpu.CompilerParams(needs_layout_passes=False),
    scratch_types=(
        [pltpu.VMEM((NP,), jnp.float32)]
        + [pltpu.VMEM((CB,), dt)
           for _ in range(RING)
           for dt in (jnp.int32, jnp.int32, jnp.float32, jnp.float32)]
        + [pltpu.SemaphoreType.DMA] * (2 * RING)
    ),
)


def kernel(ice_thickness, bedrock_elevation, meltwater_input,
           ice_sliding_velocity, node_x, node_y, length_of_link,
           node_at_link_head, node_at_link_tail, links_at_node,
           link_dirs_at_node):
    hp = jnp.pad(ice_thickness, (0, NP - N)).reshape(NP // 128, 128)
    bp = jnp.pad(bedrock_elevation, (0, NP - N)).reshape(NP // 128, 128)
    phi = pl.pallas_call(
        _phi_body,
        out_shape=jax.ShapeDtypeStruct((NP // 128, 128), jnp.float32),
    )(hp, bp).reshape(NP)
    return _sc_call(phi, node_at_link_head, node_at_link_tail, length_of_link)


# speedup vs baseline: 1.0300x; 1.0300x over previous
import jax
import jax.numpy as jnp
from jax import lax
from jax.experimental import pallas as pl
from jax.experimental.pallas import tpu as pltpu
from jax.experimental.pallas import tpu_sc as plsc

N = 100000
NP = 100096
E = 1600000
NC, NS = 2, 16
NW = NC * NS
EPW = E // NW
RING = 3
CB = 2560
_CHUNKS = []
_off = 0
while _off < EPW:
    _sz = min(CB, EPW - _off)
    _CHUNKS.append((_off, _sz))
    _off += _sz
NCHUNK = len(_CHUNKS)

PCOEF = 917.0 * 9.81
BCOEF = 1000.0 * 9.81


def _unroll(trips):
    for u in (5, 4, 3, 2):
        if trips % u == 0:
            return u
    return 1


def _phi_body(h_ref, b_ref, o_ref):
    o_ref[...] = PCOEF * h_ref[...] + BCOEF * b_ref[...]


def _sc_body(phi_hbm, head_hbm, tail_hbm, len_hbm, out_hbm, phi_v, *rest):
    c = lax.axis_index("c")
    s = lax.axis_index("s")
    wid = s * NC + c
    base = wid * EPW
    bufs = tuple(rest[4 * r:4 * r + 4] + (rest[4 * RING + 2 * r], rest[4 * RING + 2 * r + 1])
                 for r in range(RING))

    def fire_in(k):
        h, t, l, _, sin, _ = bufs[k % RING]
        off, sz = _CHUNKS[k]
        st = base + off
        return (
            pltpu.async_copy(head_hbm.at[pl.ds(st, sz)], h.at[pl.ds(0, sz)], sin),
            pltpu.async_copy(tail_hbm.at[pl.ds(st, sz)], t.at[pl.ds(0, sz)], sin),
            pltpu.async_copy(len_hbm.at[pl.ds(st, sz)], l.at[pl.ds(0, sz)], sin),
        )

    in_flight = {k: fire_in(k) for k in range(min(RING, NCHUNK))}
    pltpu.sync_copy(phi_hbm, phi_v)
    out_flight = {}
    for k in range(NCHUNK):
        h, t, l, o, _, sout = bufs[k % RING]
        off, sz = _CHUNKS[k]
        for cdesc in in_flight.pop(k):
            cdesc.wait()
        if k - RING in out_flight:
            out_flight.pop(k - RING).wait()

        @plsc.parallel_loop(0, sz, step=16, unroll=_unroll(sz // 16))
        def _gather(i):
            sl = pl.ds(i, 16)
            ph = plsc.load_gather(phi_v, [h[sl]])
            pt = plsc.load_gather(phi_v, [t[sl]])
            o[sl] = (pt - ph) / l[sl]

        st = base + off
        out_flight[k] = pltpu.async_copy(o.at[pl.ds(0, sz)], out_hbm.at[pl.ds(st, sz)], sout)
        if k + RING < NCHUNK:
            in_flight[k + RING] = fire_in(k + RING)
    for cdesc in out_flight.values():
        cdesc.wait()


_sc_call = pl.kernel(
    _sc_body,
    out_type=jax.ShapeDtypeStruct((E,), jnp.float32),
    mesh=plsc.VectorSubcoreMesh(core_axis_name="c", subcore_axis_name="s"),
    compiler_params=pltpu.CompilerParams(needs_layout_passes=False),
    scratch_types=(
        [pltpu.VMEM((NP,), jnp.float32)]
        + [pltpu.VMEM((CB,), dt)
           for _ in range(RING)
           for dt in (jnp.int32, jnp.int32, jnp.float32, jnp.float32)]
        + [pltpu.SemaphoreType.DMA] * (2 * RING)
    ),
)


def kernel(ice_thickness, bedrock_elevation, meltwater_input,
           ice_sliding_velocity, node_x, node_y, length_of_link,
           node_at_link_head, node_at_link_tail, links_at_node,
           link_dirs_at_node):
    hp = jnp.pad(ice_thickness, (0, NP - N)).reshape(NP // 128, 128)
    bp = jnp.pad(bedrock_elevation, (0, NP - N)).reshape(NP // 128, 128)
    phi = pl.pallas_call(
        _phi_body,
        out_shape=jax.ShapeDtypeStruct((NP // 128, 128), jnp.float32),
    )(hp, bp).reshape(NP)
    return _sc_call(phi, node_at_link_head, node_at_link_tail, length_of_link)
